# baseline (speedup 1.0000x reference)
import functools

import jax
import jax.numpy as jnp
from jax import lax
from jax.experimental import pallas as pl
from jax.experimental.pallas import tpu as pltpu
from jax.experimental.pallas import tpu_sc as plsc

_DIM = 64
_BATCH = 16384

_info = plsc.get_sparse_core_info()
_NC = _info.num_cores
_NS = _info.num_subcores
_NW = _NC * _NS
_BPW = _BATCH // _NW
_CHUNK = 128
_NCHUNK = _BPW // _CHUNK

_mesh = plsc.VectorSubcoreMesh(core_axis_name="c", subcore_axis_name="s")


@functools.partial(
    pl.kernel,
    mesh=_mesh,
    compiler_params=pltpu.CompilerParams(use_tc_tiling_on_sc=False),
    out_type=(
        jax.ShapeDtypeStruct((_BATCH, _DIM), jnp.float32),
        jax.ShapeDtypeStruct((_BATCH, _DIM), jnp.float32),
        jax.ShapeDtypeStruct((_BATCH, _DIM), jnp.float32),
    ),
    scratch_types=[
        pltpu.VMEM((_BPW,), jnp.int32),
        pltpu.VMEM((_BPW,), jnp.int32),
        pltpu.VMEM((_BPW,), jnp.int32),
        pltpu.VMEM((_BPW, _DIM), jnp.float32),
        pltpu.VMEM((_BPW, _DIM), jnp.float32),
        pltpu.VMEM((_BPW, _DIM), jnp.float32),
        pltpu.SemaphoreType.DMA,
        pltpu.SemaphoreType.DMA,
        pltpu.SemaphoreType.DMA,
    ],
)
def _trans_embedding(h_hbm, r_hbm, t_hbm, e_hbm, rel_hbm,
                     ho_hbm, ro_hbm, to_hbm,
                     hi_v, ri_v, ti_v, hrows_v, rrows_v, trows_v,
                     sem_h, sem_r, sem_t):
    wid = lax.axis_index("s") * _NC + lax.axis_index("c")
    base = wid * _BPW
    pltpu.sync_copy(h_hbm.at[pl.ds(base, _BPW)], hi_v)
    pltpu.sync_copy(r_hbm.at[pl.ds(base, _BPW)], ri_v)
    pltpu.sync_copy(t_hbm.at[pl.ds(base, _BPW)], ti_v)

    jobs = ((e_hbm, hi_v, hrows_v, sem_h, ho_hbm),
            (rel_hbm, ri_v, rrows_v, sem_r, ro_hbm),
            (e_hbm, ti_v, trows_v, sem_t, to_hbm))
    copies = []
    for tbl, idx_v, rows_v, sem, _ in jobs:
        for j in range(_NCHUNK):
            sl = pl.ds(j * _CHUNK, _CHUNK)
            copies.append(
                pltpu.async_copy(tbl.at[idx_v.at[sl]], rows_v.at[sl, :], sem))
    for k, (_, _, rows_v, _, out_hbm) in enumerate(jobs):
        for c in copies[k * _NCHUNK:(k + 1) * _NCHUNK]:
            c.wait()
        pltpu.sync_copy(rows_v, out_hbm.at[pl.ds(base, _BPW)])


def kernel(h, r, t, E_table, R_table):
    hh = jnp.reshape(h, (-1,)).astype(jnp.int32)
    rr = jnp.reshape(r, (-1,)).astype(jnp.int32)
    tt = jnp.reshape(t, (-1,)).astype(jnp.int32)
    return _trans_embedding(hh, rr, tt, E_table, R_table)


# speedup vs baseline: 1.5726x; 1.5726x over previous
import functools

import jax
import jax.numpy as jnp
from jax import lax
from jax.experimental import pallas as pl
from jax.experimental.pallas import tpu as pltpu
from jax.experimental.pallas import tpu_sc as plsc

_DIM = 64
_BATCH = 16384

_info = plsc.get_sparse_core_info()
_NC = _info.num_cores
_NS = _info.num_subcores
_NW = _NC * _NS
_BPW = _BATCH // _NW
_CH = 256
_NCHUNK = _BPW // _CH

_mesh = plsc.VectorSubcoreMesh(core_axis_name="c", subcore_axis_name="s")


@functools.partial(
    pl.kernel,
    mesh=_mesh,
    compiler_params=pltpu.CompilerParams(needs_layout_passes=False),
    out_type=(
        jax.ShapeDtypeStruct((_BATCH, _DIM), jnp.float32),
        jax.ShapeDtypeStruct((_BATCH, _DIM), jnp.float32),
        jax.ShapeDtypeStruct((_BATCH, _DIM), jnp.float32),
    ),
    scratch_types=[
        pltpu.VMEM((_BPW,), jnp.int32),
        pltpu.VMEM((_BPW,), jnp.int32),
        pltpu.VMEM((_BPW,), jnp.int32),
        pltpu.VMEM((_CH, _DIM), jnp.float32),
        pltpu.VMEM((_CH, _DIM), jnp.float32),
        pltpu.VMEM((_CH, _DIM), jnp.float32),
        pltpu.SemaphoreType.DMA,
        pltpu.SemaphoreType.DMA,
        pltpu.SemaphoreType.DMA,
    ],
)
def _trans_embedding(h_hbm, r_hbm, t_hbm, e_hbm, rel_hbm,
                     ho_hbm, ro_hbm, to_hbm,
                     hi_v, ri_v, ti_v, hrows_v, rrows_v, trows_v,
                     sem_h, sem_r, sem_t):
    wid = lax.axis_index("s") * _NC + lax.axis_index("c")
    base = wid * _BPW
    pltpu.sync_copy(h_hbm.at[pl.ds(base, _BPW)], hi_v)
    pltpu.sync_copy(r_hbm.at[pl.ds(base, _BPW)], ri_v)
    pltpu.sync_copy(t_hbm.at[pl.ds(base, _BPW)], ti_v)

    jobs = ((e_hbm, hi_v, hrows_v, sem_h, ho_hbm),
            (rel_hbm, ri_v, rrows_v, sem_r, ro_hbm),
            (e_hbm, ti_v, trows_v, sem_t, to_hbm))

    def fire(c, job):
        tbl, idx_v, rows_v, sem, _ = job

        def issue(g, carry):
            v = idx_v[pl.ds(c * _CH + g * 16, 16)]
            for k in range(16):
                pltpu.async_copy(tbl.at[v[k]], rows_v.at[g * 16 + k], sem)
            return carry

        lax.fori_loop(0, _CH // 16, issue, 0)

    def drain_and_write(c, job):
        tbl, _, rows_v, sem, out_hbm = job
        pltpu.make_async_copy(tbl.at[pl.ds(0, _CH)], rows_v, sem).wait()
        pltpu.sync_copy(rows_v, out_hbm.at[pl.ds(base + c * _CH, _CH)])

    for job in jobs:
        fire(0, job)
    for c in range(_NCHUNK):
        for job in jobs:
            drain_and_write(c, job)
            if c + 1 < _NCHUNK:
                fire(c + 1, job)


def kernel(h, r, t, E_table, R_table):
    hh = jnp.reshape(h, (-1,)).astype(jnp.int32)
    rr = jnp.reshape(r, (-1,)).astype(jnp.int32)
    tt = jnp.reshape(t, (-1,)).astype(jnp.int32)
    return _trans_embedding(hh, rr, tt, E_table, R_table)
